# baseline (speedup 1.0000x reference)
import jax
import jax.numpy as jnp
from jax.experimental import pallas as pl
from jax.experimental.pallas import tpu as pltpu

N = 4096
FEATS = 128
HID = 64
LABEL = 10
BLK = 512
NB = N // BLK


def _gcn_kernel(v_ref, adj_ref, w0_ref, wout_ref, bout_ref, out_ref,
                support_ref, s_ref):
    i = pl.program_id(0)

    @pl.when(i == 0)
    def _init():
        support_ref[:] = jax.lax.dot_general(
            v_ref[:], w0_ref[:], (((1,), (1,)), ((), ())),
            preferred_element_type=jnp.float32)

    h = jnp.dot(adj_ref[:], support_ref[:],
                preferred_element_type=jnp.float32)
    s_ref[0, pl.ds(i * BLK, BLK)] = jnp.sum(jnp.maximum(h, 0.0), axis=1)

    @pl.when(i == NB - 1)
    def _project():
        out_ref[:] = bout_ref[:] + jax.lax.dot_general(
            s_ref[:], wout_ref[:], (((1,), (1,)), ((), ())),
            preferred_element_type=jnp.float32)[0]


def kernel(v, adj, W0, W_out, b_out):
    return pl.pallas_call(
        _gcn_kernel,
        grid=(NB,),
        in_specs=[
            pl.BlockSpec((N, FEATS), lambda i: (0, 0)),
            pl.BlockSpec((BLK, N), lambda i: (i, 0)),
            pl.BlockSpec((HID, FEATS), lambda i: (0, 0)),
            pl.BlockSpec((LABEL, N), lambda i: (0, 0)),
            pl.BlockSpec((LABEL,), lambda i: (0,)),
        ],
        out_specs=pl.BlockSpec((LABEL,), lambda i: (0,)),
        out_shape=jax.ShapeDtypeStruct((LABEL,), jnp.float32),
        scratch_shapes=[
            pltpu.VMEM((N, HID), jnp.float32),
            pltpu.VMEM((1, N), jnp.float32),
        ],
    )(v, adj, W0.T, W_out, b_out)
